# Initial kernel scaffold; baseline (speedup 1.0000x reference)
import jax
import jax.numpy as jnp
from jax.experimental import pallas as pl


def kernel(relative_position_bias_table, relative_position_index):
    raise NotImplementedError("write your pallas kernel here")


# speedup vs baseline: 5.0286x; 5.0286x over previous
import functools

import jax
import jax.numpy as jnp
from jax import lax
from jax.experimental import pallas as pl
from jax.experimental.pallas import tpu as pltpu
from jax.experimental.pallas import tpu_sc as plsc

WH, WW = 16, 16
NUM_HEADS = 16
N = WH * WW
B = N * N
ROWS = (2 * WH - 1) * (2 * WW - 1)
TABLE_FLAT = ROWS * NUM_HEADS

_L = 16
_NW = 32
_BPW = B // _NW
_GROUPS = _BPW // _L

_mesh = plsc.VectorSubcoreMesh(core_axis_name="c", subcore_axis_name="s")


@functools.partial(
    pl.kernel,
    mesh=_mesh,
    compiler_params=pltpu.CompilerParams(needs_layout_passes=False),
    out_type=jax.ShapeDtypeStruct((NUM_HEADS, B), jnp.float32),
    scratch_types=[
        pltpu.VMEM((TABLE_FLAT,), jnp.float32),
        pltpu.VMEM((_BPW,), jnp.int32),
        pltpu.VMEM((NUM_HEADS, _BPW), jnp.float32),
    ],
)
def _bias_gather(table_hbm, idx_hbm, out_hbm, table_v, idx_v, out_v):
    wid = lax.axis_index("s") * 2 + lax.axis_index("c")
    base = wid * _BPW

    pltpu.sync_copy(table_hbm, table_v)
    pltpu.sync_copy(idx_hbm.at[pl.ds(base, _BPW)], idx_v)

    def body(g, carry):
        iv = idx_v[pl.ds(g * _L, _L)]
        rowbase = iv * NUM_HEADS
        for h in range(NUM_HEADS):
            out_v[h, pl.ds(g * _L, _L)] = plsc.load_gather(
                table_v, [rowbase + h])
        return carry

    lax.fori_loop(0, _GROUPS, body, 0)

    for h in range(NUM_HEADS):
        pltpu.sync_copy(out_v.at[h], out_hbm.at[h, pl.ds(base, _BPW)])


def kernel(relative_position_bias_table, relative_position_index):
    out = _bias_gather(relative_position_bias_table.reshape(-1),
                       relative_position_index)
    return out.reshape(1, NUM_HEADS, N, N)


# speedup vs baseline: 8.2986x; 1.6503x over previous
import functools

import jax
import jax.numpy as jnp
from jax import lax
from jax.experimental import pallas as pl
from jax.experimental.pallas import tpu as pltpu
from jax.experimental.pallas import tpu_sc as plsc

WH, WW = 16, 16
NUM_HEADS = 16
N = WH * WW
B = N * N
ROWS = (2 * WH - 1) * (2 * WW - 1)

_L = 16
_NW = 32
_BPW = B // _NW
_RPW = _BPW // N
_GROUPS = _BPW // _L

_mesh = plsc.VectorSubcoreMesh(core_axis_name="c", subcore_axis_name="s")


@functools.partial(
    pl.kernel,
    mesh=_mesh,
    compiler_params=pltpu.CompilerParams(needs_layout_passes=False),
    out_type=jax.ShapeDtypeStruct((1, NUM_HEADS, N, N), jnp.float32),
    scratch_types=[
        pltpu.VMEM((ROWS * NUM_HEADS,), jnp.float32),
        pltpu.VMEM((_BPW,), jnp.int32),
        pltpu.VMEM((NUM_HEADS, _RPW, N), jnp.float32),
    ],
)
def _bias_gather(table_hbm, idx_hbm, out_hbm, table_v, idx_v, out_v):
    wid = lax.axis_index("s") * 2 + lax.axis_index("c")
    base = wid * _BPW

    pltpu.sync_copy(table_hbm, table_v)
    pltpu.sync_copy(idx_hbm.at[pl.ds(base, _BPW)], idx_v)

    @plsc.parallel_loop(0, _GROUPS, unroll=2)
    def _(g):
        iv = idx_v[pl.ds(g * _L, _L)]
        r = g // _L
        c = (g % _L) * _L
        rowbase = iv * NUM_HEADS
        vals = [plsc.load_gather(table_v, [rowbase + h])
                for h in range(NUM_HEADS)]
        for h in range(NUM_HEADS):
            out_v[h, r, pl.ds(c, _L)] = vals[h]

    for h in range(NUM_HEADS):
        pltpu.sync_copy(out_v.at[h],
                        out_hbm.at[0, h, pl.ds(wid * _RPW, _RPW), :])


def kernel(relative_position_bias_table, relative_position_index):
    return _bias_gather(relative_position_bias_table.reshape(-1),
                        relative_position_index)
